# baseline (speedup 1.0000x reference)
import functools

import jax
import jax.numpy as jnp
from jax import lax
from jax.experimental import pallas as pl
from jax.experimental.pallas import tpu as pltpu
from jax.experimental.pallas import tpu_sc as plsc

B = 16384
D_TABLE = 56
D_SHARED = 8
D_OUT = 64
NC = 2
NS = 16
NW = NC * NS
ROWS_PER_W = B // NW

_mesh = plsc.VectorSubcoreMesh(core_axis_name="c", subcore_axis_name="s")


@functools.partial(
    pl.kernel,
    mesh=_mesh,
    out_type=jax.ShapeDtypeStruct((B, D_TABLE), jnp.float32),
    scratch_types=[
        pltpu.VMEM((ROWS_PER_W // 16, 16), jnp.int32),
        pltpu.VMEM_SHARED((B // NC, D_TABLE), jnp.float32),
        pltpu.SemaphoreType.DMA((8,)),
    ],
)
def _emb_gather(x_hbm, table_hbm, out_hbm, idx_v, rows_v, sem):
    wid = lax.axis_index("s") * NC + lax.axis_index("c")
    base = wid * ROWS_PER_W

    ng = ROWS_PER_W // 16
    pltpu.sync_copy(x_hbm.at[pl.ds(wid * ng, ng)], idx_v)

    sid = lax.axis_index("s")
    tbase = sid * ROWS_PER_W

    def _issue(g, _):
        vec = idx_v[g]
        for k in range(16):
            s = vec[k]
            pltpu.async_copy(
                table_hbm.at[s], rows_v.at[tbase + g * 16 + k], sem.at[k % 8]
            )
        return 0

    lax.fori_loop(0, ROWS_PER_W // 16, _issue, 0)

    for k in range(8):
        pltpu.make_async_copy(
            table_hbm.at[pl.ds(0, ROWS_PER_W // 8)],
            rows_v.at[pl.ds(0, ROWS_PER_W // 8)],
            sem.at[k],
        ).wait()

    pltpu.sync_copy(
        rows_v.at[pl.ds(tbase, ROWS_PER_W)], out_hbm.at[pl.ds(base, ROWS_PER_W)]
    )


def kernel(x, table, shared):
    rows = _emb_gather(x.astype(jnp.int32).reshape(B // 16, 16), table)
    shared_exp = jnp.broadcast_to(shared.reshape(1, D_SHARED), (B, D_SHARED))
    out = jnp.concatenate((rows, shared_exp), axis=-1)
    return out.reshape(B, 1, D_OUT)


# speedup vs baseline: 1.0813x; 1.0813x over previous
import functools

import jax
import jax.numpy as jnp
from jax import lax
from jax.experimental import pallas as pl
from jax.experimental.pallas import tpu as pltpu
from jax.experimental.pallas import tpu_sc as plsc

B = 16384
D_TABLE = 56
D_SHARED = 8
D_OUT = 64
NC = 2
NS = 16
NW = NC * NS
ROWS_PER_W = B // NW

_mesh = plsc.VectorSubcoreMesh(core_axis_name="c", subcore_axis_name="s")


@functools.partial(
    pl.kernel,
    mesh=_mesh,
    out_type=jax.ShapeDtypeStruct((B, D_TABLE), jnp.float32),
    scratch_types=[
        pltpu.VMEM((ROWS_PER_W // 16, 16), jnp.int32),
        pltpu.VMEM((ROWS_PER_W, D_TABLE), jnp.float32),
        pltpu.SemaphoreType.DMA((8,)),
    ],
)
def _emb_gather(x_hbm, table_hbm, out_hbm, idx_v, rows_v, sem):
    wid = lax.axis_index("s") * NC + lax.axis_index("c")
    base = wid * ROWS_PER_W

    ng = ROWS_PER_W // 16
    pltpu.sync_copy(x_hbm.at[pl.ds(wid * ng, ng)], idx_v)

    def _issue(g, _):
        vec = idx_v[g]
        for k in range(16):
            s = vec[k]
            pltpu.async_copy(table_hbm.at[s], rows_v.at[g * 16 + k], sem.at[k % 8])
        return 0

    lax.fori_loop(0, ROWS_PER_W // 16, _issue, 0)

    for k in range(8):
        pltpu.make_async_copy(
            table_hbm.at[pl.ds(0, ROWS_PER_W // 8)],
            rows_v.at[pl.ds(0, ROWS_PER_W // 8)],
            sem.at[k],
        ).wait()

    pltpu.sync_copy(rows_v, out_hbm.at[pl.ds(base, ROWS_PER_W)])


def kernel(x, table, shared):
    rows = _emb_gather(x.astype(jnp.int32).reshape(B // 16, 16), table)
    shared_exp = jnp.broadcast_to(shared.reshape(1, D_SHARED), (B, D_SHARED))
    out = jnp.concatenate((rows, shared_exp), axis=-1)
    return out.reshape(B, 1, D_OUT)
